# baseline (speedup 1.0000x reference)
import functools

import jax
import jax.numpy as jnp
from jax import lax
from jax.experimental import pallas as pl
from jax.experimental.pallas import tpu as pltpu
from jax.experimental.pallas import tpu_sc as plsc

N = 10000
F_IN = 128
H = 128
C = 16
KNN = 5
E = 320000

NC = 2
NS = 16
NW = NC * NS
EB = 128
ACC = 10240
TRASH = 10016
BM = 512
KBM = 256
NPADC = 10240


def _cdiv(a, b):
    return (a + b - 1) // b


def _prep_body(x_ref, w1_ref, xn_ref, xw_ref):
    x = x_ref[...]
    nrm = jnp.sqrt(jnp.sum(x * x, axis=1, keepdims=True))
    xn_ref[...] = x / (nrm + 1e-12)
    xw_ref[...] = jnp.dot(x, w1_ref[...], preferred_element_type=jnp.float32)


def _prep(x, W1):
    g = _cdiv(N, BM)
    return pl.pallas_call(
        _prep_body,
        grid=(g,),
        in_specs=[pl.BlockSpec((BM, F_IN), lambda i: (i, 0)),
                  pl.BlockSpec((F_IN, H), lambda i: (0, 0))],
        out_specs=[pl.BlockSpec((BM, F_IN), lambda i: (i, 0)),
                   pl.BlockSpec((BM, H), lambda i: (i, 0))],
        out_shape=[jax.ShapeDtypeStruct((N, F_IN), jnp.float32),
                   jax.ShapeDtypeStruct((N, H), jnp.float32)],
    )(x, W1)


def _knn_body(xn_ref, xnt_ref, out_ref):
    i = pl.program_id(0)
    a = xn_ref[...]
    s = jnp.dot(a, xnt_ref[...], preferred_element_type=jnp.float32)
    rows = i * KBM + lax.broadcasted_iota(jnp.int32, s.shape, 0)
    cols = lax.broadcasted_iota(jnp.int32, s.shape, 1)
    s = jnp.where((cols == rows) | (cols >= N), -1e30, s)
    lane = lax.broadcasted_iota(jnp.int32, (KBM, 128), 1)
    res = jnp.zeros((KBM, 128), jnp.int32)
    for j in range(KNN):
        m = jnp.max(s, axis=1, keepdims=True)
        arg = jnp.min(jnp.where(s == m, cols, jnp.int32(2 ** 30)), axis=1,
                      keepdims=True)
        res = jnp.where(lane == j, arg, res)
        s = jnp.where(cols == arg, -1e30, s)
    out_ref[...] = res


def _knn(xn, xnt_pad):
    g = _cdiv(N, KBM)
    return pl.pallas_call(
        _knn_body,
        grid=(g,),
        in_specs=[pl.BlockSpec((KBM, F_IN), lambda i: (i, 0)),
                  pl.BlockSpec((F_IN, NPADC), lambda i: (0, 0))],
        out_specs=pl.BlockSpec((KBM, 128), lambda i: (i, 0)),
        out_shape=jax.ShapeDtypeStruct((N, 128), jnp.int32),
    )(xn, xnt_pad)


def _dinv(dref):
    d = dref[0, :, 0:1] + dref[1, :, 0:1]
    return jnp.where(d > 0, 1.0 / jnp.sqrt(d), 0.0)


HH = H // 2


def _scale_body(xw_ref, dg_ref, dk_ref, xsg_lo, xsg_hi, xsk_lo, xsk_hi):
    xw = xw_ref[...]
    xsg = xw * _dinv(dg_ref)
    xsk = xw * _dinv(dk_ref)
    xsg_lo[...] = xsg[:, :HH]
    xsg_hi[...] = xsg[:, HH:]
    xsk_lo[...] = xsk[:, :HH]
    xsk_hi[...] = xsk[:, HH:]


def _scale(xw1, degg, degk):
    g = _cdiv(N, BM)
    dspec = pl.BlockSpec((2, BM, C), lambda i: (0, i, 0))
    ospec = pl.BlockSpec((BM, HH), lambda i: (i, 0))
    return pl.pallas_call(
        _scale_body,
        grid=(g,),
        in_specs=[pl.BlockSpec((BM, H), lambda i: (i, 0)), dspec, dspec],
        out_specs=[ospec, ospec, ospec, ospec],
        out_shape=[jax.ShapeDtypeStruct((N, HH), jnp.float32)] * 4,
    )(xw1, degg, degk)


def _assemble_body(aglo_ref, aghi_ref, aklo_ref, akhi_ref, xw_ref, dg_ref,
                   dk_ref, b1_ref, w2_ref, rw2_ref, xs2g_ref, xs2k_ref):
    dig = _dinv(dg_ref)
    dik = _dinv(dk_ref)
    b1 = b1_ref[...]
    ag = jnp.concatenate([aglo_ref[0] + aglo_ref[1],
                          aghi_ref[0] + aghi_ref[1]], axis=1)
    ak = jnp.concatenate([aklo_ref[0] + aklo_ref[1],
                          akhi_ref[0] + akhi_ref[1]], axis=1)
    h1 = jnp.maximum(ag * dig + b1, 0.0)
    h12 = jnp.maximum(ak * dik + b1, 0.0)
    h13 = jnp.maximum(xw_ref[...] + b1, 0.0)
    w2 = w2_ref[...]
    rw2 = (jnp.dot(h1, w2[0:H], preferred_element_type=jnp.float32)
           + jnp.dot(h12, w2[H:2 * H], preferred_element_type=jnp.float32)
           + jnp.dot(h13, w2[2 * H:3 * H], preferred_element_type=jnp.float32))
    rw2_ref[...] = rw2
    xs2g_ref[...] = rw2 * dig
    xs2k_ref[...] = rw2 * dik


def _assemble(aglo, aghi, aklo, akhi, xw1, degg, degk, b1r, W2):
    g = _cdiv(N, BM)
    aspec = pl.BlockSpec((2, BM, HH), lambda i: (0, i, 0))
    dspec = pl.BlockSpec((2, BM, C), lambda i: (0, i, 0))
    ospec = pl.BlockSpec((BM, C), lambda i: (i, 0))
    return pl.pallas_call(
        _assemble_body,
        grid=(g,),
        in_specs=[aspec, aspec, aspec, aspec,
                  pl.BlockSpec((BM, H), lambda i: (i, 0)),
                  dspec, dspec,
                  pl.BlockSpec((1, H), lambda i: (0, 0)),
                  pl.BlockSpec((3 * H, C), lambda i: (0, 0))],
        out_specs=[ospec, ospec, ospec],
        out_shape=[jax.ShapeDtypeStruct((N, C), jnp.float32)] * 3,
    )(aglo, aghi, aklo, akhi, xw1, degg, degk, b1r, W2)


def _final_body(ag_ref, ak_ref, rw2_ref, dg_ref, dk_ref, b2_ref, wl_ref,
                bl_ref, out_ref):
    dig = _dinv(dg_ref)
    dik = _dinv(dk_ref)
    b2 = b2_ref[...]
    h2 = (ag_ref[0] + ag_ref[1]) * dig + b2
    h22 = (ak_ref[0] + ak_ref[1]) * dik + b2
    h23 = rw2_ref[...] + b2
    wl = wl_ref[...]
    f = (jnp.dot(h2, wl[0:C], preferred_element_type=jnp.float32)
         + jnp.dot(h22, wl[C:2 * C], preferred_element_type=jnp.float32)
         + jnp.dot(h23, wl[2 * C:3 * C], preferred_element_type=jnp.float32)
         + bl_ref[...])
    m = jnp.max(f, axis=1, keepdims=True)
    e = jnp.exp(f - m)
    lse = jnp.log(jnp.sum(e, axis=1, keepdims=True))
    out_ref[...] = f - m - lse


def _final(acc2g, acc2k, rw2, degg, degk, b2r, Wlin, blinr):
    g = _cdiv(N, BM)
    aspec = pl.BlockSpec((2, BM, C), lambda i: (0, i, 0))
    return pl.pallas_call(
        _final_body,
        grid=(g,),
        in_specs=[aspec, aspec, pl.BlockSpec((BM, C), lambda i: (i, 0)),
                  aspec, aspec,
                  pl.BlockSpec((1, C), lambda i: (0, 0)),
                  pl.BlockSpec((3 * C, C), lambda i: (0, 0)),
                  pl.BlockSpec((1, C), lambda i: (0, 0))],
        out_specs=pl.BlockSpec((BM, C), lambda i: (i, 0)),
        out_shape=jax.ShapeDtypeStruct((N, C), jnp.float32),
    )(acc2g, acc2k, rw2, degg, degk, b2r, Wlin, blinr)


@functools.lru_cache(maxsize=None)
def _mesh():
    return plsc.VectorSubcoreMesh(core_axis_name="c", subcore_axis_name="s")


_RPT = ACC // NS
_RCH = _RPT // 128


@functools.lru_cache(maxsize=None)
def _sc_hist(nbt):
    np_ = nbt - 1

    @functools.partial(
        pl.kernel,
        mesh=_mesh(),
        compiler_params=pltpu.CompilerParams(use_tc_tiling_on_sc=False),
        out_type=jax.ShapeDtypeStruct((NC, ACC, C), jnp.float32),
        scratch_types=[pltpu.VMEM((nbt, EB), jnp.int32),
                       pltpu.VMEM((128, C), jnp.float32),
                       pltpu.VMEM((128, C), jnp.float32),
                       pltpu.VMEM_SHARED((ACC, C), jnp.float32),
                       pltpu.SemaphoreType.DMA],
    )
    def k(dst_hbm, ones_hbm, zeros_hbm, out_hbm, dst_v, ones_v, buf_v, acc, sem):
        c = lax.axis_index("c")
        s = lax.axis_index("s")
        pltpu.sync_copy(dst_hbm.at[c, s], dst_v)
        pltpu.sync_copy(ones_hbm, ones_v)
        pltpu.sync_copy(zeros_hbm, buf_v)
        for t in range(_RCH):
            pltpu.sync_copy(buf_v, acc.at[pl.ds(s * _RPT + t * 128, 128)])
        plsc.subcore_barrier()

        def fire(j, carry):
            pltpu.async_copy(ones_v, acc.at[dst_v.at[j]], sem, add=True)
            return carry

        lax.fori_loop(0, np_, fire, 0)

        def drain(j, carry):
            pltpu.make_async_copy(ones_hbm, ones_v, sem).wait()
            return carry

        lax.fori_loop(0, np_, drain, 0)
        plsc.subcore_barrier()
        for t in range(_RCH):
            pltpu.sync_copy(acc.at[pl.ds(s * _RPT + t * 128, 128)], buf_v)
            pltpu.sync_copy(buf_v, out_hbm.at[c, pl.ds(s * _RPT + t * 128, 128)])

    return k


@functools.lru_cache(maxsize=None)
def _sc_gs(nbt, d):
    np_ = nbt - 1
    nh = np_ // 2

    @functools.partial(
        pl.kernel,
        mesh=_mesh(),
        compiler_params=pltpu.CompilerParams(use_tc_tiling_on_sc=False),
        out_type=jax.ShapeDtypeStruct((NC, ACC, d), jnp.float32),
        scratch_types=[pltpu.VMEM((nbt, EB), jnp.int32),
                       pltpu.VMEM((nbt, EB), jnp.int32),
                       pltpu.VMEM((EB, d), jnp.float32),
                       pltpu.VMEM((EB, d), jnp.float32),
                       pltpu.VMEM((128, d), jnp.float32),
                       pltpu.VMEM_SHARED((ACC, d), jnp.float32),
                       pltpu.SemaphoreType.DMA,
                       pltpu.SemaphoreType.DMA],
    )
    def k(tab_hbm, src_hbm, dst_hbm, zeros_hbm, out_hbm,
          src_v, dst_v, rows0, rows1, buf_v, acc, sg0, sg1):
        c = lax.axis_index("c")
        s = lax.axis_index("s")
        pltpu.sync_copy(src_hbm.at[c, s], src_v)
        pltpu.sync_copy(dst_hbm.at[c, s], dst_v)
        pltpu.sync_copy(zeros_hbm, buf_v)
        for t in range(_RCH):
            pltpu.sync_copy(buf_v, acc.at[pl.ds(s * _RPT + t * 128, 128)])
        plsc.subcore_barrier()

        pltpu.async_copy(tab_hbm.at[src_v.at[0]], rows0, sg0)

        def body(i, carry):
            j = 2 * i
            pltpu.make_async_copy(tab_hbm.at[src_v.at[0]], rows0, sg0).wait()
            pltpu.async_copy(tab_hbm.at[src_v.at[j + 1]], rows1, sg1)
            pltpu.sync_copy(rows0, acc.at[dst_v.at[j]], add=True)
            pltpu.make_async_copy(tab_hbm.at[src_v.at[0]], rows1, sg1).wait()
            pltpu.async_copy(tab_hbm.at[src_v.at[j + 2]], rows0, sg0)
            pltpu.sync_copy(rows1, acc.at[dst_v.at[j + 1]], add=True)
            return carry

        lax.fori_loop(0, nh, body, 0)
        pltpu.make_async_copy(tab_hbm.at[src_v.at[0]], rows0, sg0).wait()
        plsc.subcore_barrier()
        for t in range(_RCH):
            pltpu.sync_copy(acc.at[pl.ds(s * _RPT + t * 128, 128)], buf_v)
            pltpu.sync_copy(buf_v, out_hbm.at[c, pl.ds(s * _RPT + t * 128, 128)])

    return k


def _edge_blocks(idx, fill, np_):
    tot = NW * np_ * EB
    pad = tot - idx.shape[0]
    idx = jnp.concatenate(
        [idx, jnp.full((pad,), fill, jnp.int32)]) if pad else idx
    arr = idx.reshape(NW, np_ * EB)
    arr = jnp.concatenate(
        [arr, jnp.full((NW, EB), fill, jnp.int32)], axis=1)
    return arr.reshape(NC, NS, np_ + 1, EB)


def kernel(x, edge_index, W1, b1, W2, b2, Wlin, blin):
    f32 = jnp.float32
    np_g = 80
    np_k = 14
    nbt_g = np_g + 1
    nbt_k = np_k + 1

    xn, xw1 = _prep(x, W1)
    xnt_pad = jnp.concatenate(
        [xn.T, jnp.zeros((F_IN, NPADC - N), f32)], axis=1)
    nbr = _knn(xn, xnt_pad)[:, :KNN]

    src_g = edge_index[0]
    dst_g = jnp.where(edge_index[0] == edge_index[1], TRASH, edge_index[1])
    srcg3 = _edge_blocks(src_g, 0, np_g)
    dstg3 = _edge_blocks(dst_g, TRASH, np_g)
    src_k = jnp.broadcast_to(jnp.arange(N, dtype=jnp.int32)[:, None],
                             (N, KNN)).reshape(-1)
    dst_k = nbr.reshape(-1)
    srck3 = _edge_blocks(src_k, 0, np_k)
    dstk3 = _edge_blocks(dst_k, TRASH, np_k)

    ones_c = jnp.ones((128, C), f32)
    zeros_c = jnp.zeros((128, C), f32)
    zeros_h = jnp.zeros((128, HH), f32)

    degg = _sc_hist(nbt_g)(dstg3, ones_c, zeros_c)
    degk = _sc_hist(nbt_k)(dstk3, ones_c, zeros_c)

    xsg_lo, xsg_hi, xsk_lo, xsk_hi = _scale(xw1, degg, degk)
    gs_h = _sc_gs(nbt_g, HH)
    gs_hk = _sc_gs(nbt_k, HH)
    aglo = gs_h(xsg_lo, srcg3, dstg3, zeros_h)
    aghi = gs_h(xsg_hi, srcg3, dstg3, zeros_h)
    aklo = gs_hk(xsk_lo, srck3, dstk3, zeros_h)
    akhi = gs_hk(xsk_hi, srck3, dstk3, zeros_h)

    rw2, xs2g, xs2k = _assemble(aglo, aghi, aklo, akhi, xw1, degg, degk,
                                b1.reshape(1, H), W2)
    acc2g = _sc_gs(nbt_g, C)(xs2g, srcg3, dstg3, zeros_c)
    acc2k = _sc_gs(nbt_k, C)(xs2k, srck3, dstk3, zeros_c)

    return _final(acc2g, acc2k, rw2, degg, degk,
                  b2.reshape(1, C), Wlin, blin.reshape(1, C))


# speedup vs baseline: 1.0039x; 1.0039x over previous
import functools

import jax
import jax.numpy as jnp
from jax import lax
from jax.experimental import pallas as pl
from jax.experimental.pallas import tpu as pltpu
from jax.experimental.pallas import tpu_sc as plsc

N = 10000
F_IN = 128
H = 128
C = 16
KNN = 5
E = 320000

NC = 2
NS = 16
NW = NC * NS
EB = 128
ACC = 10240
TRASH = 10016
BM = 512
KBM = 256
NPADC = 10240


def _cdiv(a, b):
    return (a + b - 1) // b


def _prep_body(x_ref, w1_ref, xn_ref, xw_ref):
    x = x_ref[...]
    nrm = jnp.sqrt(jnp.sum(x * x, axis=1, keepdims=True))
    xn_ref[...] = x / (nrm + 1e-12)
    xw_ref[...] = jnp.dot(x, w1_ref[...], preferred_element_type=jnp.float32)


def _prep(x, W1):
    g = _cdiv(N, BM)
    return pl.pallas_call(
        _prep_body,
        grid=(g,),
        in_specs=[pl.BlockSpec((BM, F_IN), lambda i: (i, 0)),
                  pl.BlockSpec((F_IN, H), lambda i: (0, 0))],
        out_specs=[pl.BlockSpec((BM, F_IN), lambda i: (i, 0)),
                   pl.BlockSpec((BM, H), lambda i: (i, 0))],
        out_shape=[jax.ShapeDtypeStruct((N, F_IN), jnp.float32),
                   jax.ShapeDtypeStruct((N, H), jnp.float32)],
    )(x, W1)


def _knn_body(xn_ref, xnt_ref, out_ref):
    i = pl.program_id(0)
    a = xn_ref[...]
    s = jnp.dot(a, xnt_ref[...], preferred_element_type=jnp.float32)
    rows = i * KBM + lax.broadcasted_iota(jnp.int32, s.shape, 0)
    cols = lax.broadcasted_iota(jnp.int32, s.shape, 1)
    s = jnp.where((cols == rows) | (cols >= N), -1e30, s)
    lane = lax.broadcasted_iota(jnp.int32, (KBM, 128), 1)
    res = jnp.zeros((KBM, 128), jnp.int32)
    for j in range(KNN):
        m = jnp.max(s, axis=1, keepdims=True)
        arg = jnp.min(jnp.where(s == m, cols, jnp.int32(2 ** 30)), axis=1,
                      keepdims=True)
        res = jnp.where(lane == j, arg, res)
        s = jnp.where(cols == arg, -1e30, s)
    out_ref[...] = res


def _knn(xn, xnt_pad):
    g = _cdiv(N, KBM)
    return pl.pallas_call(
        _knn_body,
        grid=(g,),
        in_specs=[pl.BlockSpec((KBM, F_IN), lambda i: (i, 0)),
                  pl.BlockSpec((F_IN, NPADC), lambda i: (0, 0))],
        out_specs=pl.BlockSpec((KBM, 128), lambda i: (i, 0)),
        out_shape=jax.ShapeDtypeStruct((N, 128), jnp.int32),
    )(xn, xnt_pad)


def _dinv(dref):
    d = dref[0, :, 0:1] + dref[1, :, 0:1]
    return jnp.where(d > 0, 1.0 / jnp.sqrt(d), 0.0)


HH = H // 2


def _scale_body(xw_ref, dg_ref, dk_ref, xsg_lo, xsg_hi, xsk_lo, xsk_hi):
    xw = xw_ref[...]
    xsg = xw * _dinv(dg_ref)
    xsk = xw * _dinv(dk_ref)
    xsg_lo[...] = xsg[:, :HH]
    xsg_hi[...] = xsg[:, HH:]
    xsk_lo[...] = xsk[:, :HH]
    xsk_hi[...] = xsk[:, HH:]


def _scale(xw1, degg, degk):
    g = _cdiv(N, BM)
    dspec = pl.BlockSpec((2, BM, C), lambda i: (0, i, 0))
    ospec = pl.BlockSpec((BM, HH), lambda i: (i, 0))
    return pl.pallas_call(
        _scale_body,
        grid=(g,),
        in_specs=[pl.BlockSpec((BM, H), lambda i: (i, 0)), dspec, dspec],
        out_specs=[ospec, ospec, ospec, ospec],
        out_shape=[jax.ShapeDtypeStruct((N, HH), jnp.float32)] * 4,
    )(xw1, degg, degk)


def _assemble_body(aglo_ref, aghi_ref, aklo_ref, akhi_ref, xw_ref, dg_ref,
                   dk_ref, b1_ref, w2_ref, rw2_ref, xs2g_ref, xs2k_ref):
    dig = _dinv(dg_ref)
    dik = _dinv(dk_ref)
    b1 = b1_ref[...]
    ag = jnp.concatenate([aglo_ref[0] + aglo_ref[1],
                          aghi_ref[0] + aghi_ref[1]], axis=1)
    ak = jnp.concatenate([aklo_ref[0] + aklo_ref[1],
                          akhi_ref[0] + akhi_ref[1]], axis=1)
    h1 = jnp.maximum(ag * dig + b1, 0.0)
    h12 = jnp.maximum(ak * dik + b1, 0.0)
    h13 = jnp.maximum(xw_ref[...] + b1, 0.0)
    w2 = w2_ref[...]
    rw2 = (jnp.dot(h1, w2[0:H], preferred_element_type=jnp.float32)
           + jnp.dot(h12, w2[H:2 * H], preferred_element_type=jnp.float32)
           + jnp.dot(h13, w2[2 * H:3 * H], preferred_element_type=jnp.float32))
    rw2_ref[...] = rw2
    xs2g_ref[...] = rw2 * dig
    xs2k_ref[...] = rw2 * dik


def _assemble(aglo, aghi, aklo, akhi, xw1, degg, degk, b1r, W2):
    g = _cdiv(N, BM)
    aspec = pl.BlockSpec((2, BM, HH), lambda i: (0, i, 0))
    dspec = pl.BlockSpec((2, BM, C), lambda i: (0, i, 0))
    ospec = pl.BlockSpec((BM, C), lambda i: (i, 0))
    return pl.pallas_call(
        _assemble_body,
        grid=(g,),
        in_specs=[aspec, aspec, aspec, aspec,
                  pl.BlockSpec((BM, H), lambda i: (i, 0)),
                  dspec, dspec,
                  pl.BlockSpec((1, H), lambda i: (0, 0)),
                  pl.BlockSpec((3 * H, C), lambda i: (0, 0))],
        out_specs=[ospec, ospec, ospec],
        out_shape=[jax.ShapeDtypeStruct((N, C), jnp.float32)] * 3,
    )(aglo, aghi, aklo, akhi, xw1, degg, degk, b1r, W2)


def _final_body(ag_ref, ak_ref, rw2_ref, dg_ref, dk_ref, b2_ref, wl_ref,
                bl_ref, out_ref):
    dig = _dinv(dg_ref)
    dik = _dinv(dk_ref)
    b2 = b2_ref[...]
    h2 = (ag_ref[0] + ag_ref[1]) * dig + b2
    h22 = (ak_ref[0] + ak_ref[1]) * dik + b2
    h23 = rw2_ref[...] + b2
    wl = wl_ref[...]
    f = (jnp.dot(h2, wl[0:C], preferred_element_type=jnp.float32)
         + jnp.dot(h22, wl[C:2 * C], preferred_element_type=jnp.float32)
         + jnp.dot(h23, wl[2 * C:3 * C], preferred_element_type=jnp.float32)
         + bl_ref[...])
    m = jnp.max(f, axis=1, keepdims=True)
    e = jnp.exp(f - m)
    lse = jnp.log(jnp.sum(e, axis=1, keepdims=True))
    out_ref[...] = f - m - lse


def _final(acc2g, acc2k, rw2, degg, degk, b2r, Wlin, blinr):
    g = _cdiv(N, BM)
    aspec = pl.BlockSpec((2, BM, C), lambda i: (0, i, 0))
    return pl.pallas_call(
        _final_body,
        grid=(g,),
        in_specs=[aspec, aspec, pl.BlockSpec((BM, C), lambda i: (i, 0)),
                  aspec, aspec,
                  pl.BlockSpec((1, C), lambda i: (0, 0)),
                  pl.BlockSpec((3 * C, C), lambda i: (0, 0)),
                  pl.BlockSpec((1, C), lambda i: (0, 0))],
        out_specs=pl.BlockSpec((BM, C), lambda i: (i, 0)),
        out_shape=jax.ShapeDtypeStruct((N, C), jnp.float32),
    )(acc2g, acc2k, rw2, degg, degk, b2r, Wlin, blinr)


@functools.lru_cache(maxsize=None)
def _mesh():
    return plsc.VectorSubcoreMesh(core_axis_name="c", subcore_axis_name="s")


_RPT = ACC // NS
_RCH = _RPT // 128


@functools.lru_cache(maxsize=None)
def _sc_hist(nbt):
    np_ = nbt - 1

    @functools.partial(
        pl.kernel,
        mesh=_mesh(),
        compiler_params=pltpu.CompilerParams(use_tc_tiling_on_sc=False),
        out_type=jax.ShapeDtypeStruct((NC, ACC, C), jnp.float32),
        scratch_types=[pltpu.VMEM((nbt, EB), jnp.int32),
                       pltpu.VMEM((128, C), jnp.float32),
                       pltpu.VMEM((128, C), jnp.float32),
                       pltpu.VMEM_SHARED((ACC, C), jnp.float32),
                       pltpu.SemaphoreType.DMA],
    )
    def k(dst_hbm, ones_hbm, zeros_hbm, out_hbm, dst_v, ones_v, buf_v, acc, sem):
        c = lax.axis_index("c")
        s = lax.axis_index("s")
        pltpu.sync_copy(dst_hbm.at[c, s], dst_v)
        pltpu.sync_copy(ones_hbm, ones_v)
        pltpu.sync_copy(zeros_hbm, buf_v)
        for t in range(_RCH):
            pltpu.sync_copy(buf_v, acc.at[pl.ds(s * _RPT + t * 128, 128)])
        plsc.subcore_barrier()

        def fire(j, carry):
            pltpu.async_copy(ones_v, acc.at[dst_v.at[j]], sem, add=True)
            return carry

        lax.fori_loop(0, np_, fire, 0)

        def drain(j, carry):
            pltpu.make_async_copy(ones_hbm, ones_v, sem).wait()
            return carry

        lax.fori_loop(0, np_, drain, 0)
        plsc.subcore_barrier()
        for t in range(_RCH):
            pltpu.sync_copy(acc.at[pl.ds(s * _RPT + t * 128, 128)], buf_v)
            pltpu.sync_copy(buf_v, out_hbm.at[c, pl.ds(s * _RPT + t * 128, 128)])

    return k


@functools.lru_cache(maxsize=None)
def _sc_gs(nbt, d):
    np_ = nbt - 1
    nh = np_ // 2

    @functools.partial(
        pl.kernel,
        mesh=_mesh(),
        compiler_params=pltpu.CompilerParams(use_tc_tiling_on_sc=False),
        out_type=jax.ShapeDtypeStruct((NC, ACC, d), jnp.float32),
        scratch_types=[pltpu.VMEM((nbt, EB), jnp.int32),
                       pltpu.VMEM((nbt, EB), jnp.int32),
                       pltpu.VMEM((EB, d), jnp.float32),
                       pltpu.VMEM((EB, d), jnp.float32),
                       pltpu.VMEM((128, d), jnp.float32),
                       pltpu.VMEM_SHARED((ACC, d), jnp.float32),
                       pltpu.SemaphoreType.DMA,
                       pltpu.SemaphoreType.DMA],
    )
    def k(tab_hbm, src_hbm, dst_hbm, zeros_hbm, out_hbm,
          src_v, dst_v, rows0, rows1, buf_v, acc, sg0, sg1):
        c = lax.axis_index("c")
        s = lax.axis_index("s")
        pltpu.sync_copy(src_hbm.at[c, s], src_v)
        pltpu.sync_copy(dst_hbm.at[c, s], dst_v)
        pltpu.sync_copy(zeros_hbm, buf_v)
        for t in range(_RCH):
            pltpu.sync_copy(buf_v, acc.at[pl.ds(s * _RPT + t * 128, 128)])
        plsc.subcore_barrier()

        pltpu.async_copy(tab_hbm.at[src_v.at[0]], rows0, sg0)

        def body(i, carry):
            j = 2 * i
            pltpu.make_async_copy(tab_hbm.at[src_v.at[0]], rows0, sg0).wait()
            pltpu.async_copy(tab_hbm.at[src_v.at[j + 1]], rows1, sg1)
            pltpu.sync_copy(rows0, acc.at[dst_v.at[j]], add=True)
            pltpu.make_async_copy(tab_hbm.at[src_v.at[0]], rows1, sg1).wait()
            pltpu.async_copy(tab_hbm.at[src_v.at[j + 2]], rows0, sg0)
            pltpu.sync_copy(rows1, acc.at[dst_v.at[j + 1]], add=True)
            return carry

        lax.fori_loop(0, nh, body, 0)
        pltpu.make_async_copy(tab_hbm.at[src_v.at[0]], rows0, sg0).wait()
        plsc.subcore_barrier()
        for t in range(_RCH):
            pltpu.sync_copy(acc.at[pl.ds(s * _RPT + t * 128, 128)], buf_v)
            pltpu.sync_copy(buf_v, out_hbm.at[c, pl.ds(s * _RPT + t * 128, 128)])

    return k


def _trash(n):
    return N + (jnp.arange(n, dtype=jnp.int32) % (ACC - N))


def _edge_blocks(idx, np_, trash_fill):
    tot = NW * np_ * EB
    pad = tot - idx.shape[0]
    fill = _trash(pad) if trash_fill else jnp.zeros((pad,), jnp.int32)
    extra = (_trash(NW * EB) if trash_fill
             else jnp.zeros((NW * EB,), jnp.int32)).reshape(NW, EB)
    arr = jnp.concatenate([idx, fill]).reshape(NW, np_ * EB)
    arr = jnp.concatenate([arr, extra], axis=1)
    return arr.reshape(NC, NS, np_ + 1, EB)


def kernel(x, edge_index, W1, b1, W2, b2, Wlin, blin):
    f32 = jnp.float32
    np_g = 80
    np_k = 14
    nbt_g = np_g + 1
    nbt_k = np_k + 1

    xn, xw1 = _prep(x, W1)
    xnt_pad = jnp.concatenate(
        [xn.T, jnp.zeros((F_IN, NPADC - N), f32)], axis=1)
    nbr = _knn(xn, xnt_pad)[:, :KNN]

    src_g = edge_index[0]
    dst_g = jnp.where(edge_index[0] == edge_index[1],
                      N + (edge_index[0] % (ACC - N)), edge_index[1])
    srcg3 = _edge_blocks(src_g, np_g, False)
    dstg3 = _edge_blocks(dst_g, np_g, True)
    src_k = jnp.broadcast_to(jnp.arange(N, dtype=jnp.int32)[:, None],
                             (N, KNN)).reshape(-1)
    dst_k = nbr.reshape(-1)
    srck3 = _edge_blocks(src_k, np_k, False)
    dstk3 = _edge_blocks(dst_k, np_k, True)

    ones_c = jnp.ones((128, C), f32)
    zeros_c = jnp.zeros((128, C), f32)
    zeros_h = jnp.zeros((128, HH), f32)

    degg = _sc_hist(nbt_g)(dstg3, ones_c, zeros_c)
    degk = _sc_hist(nbt_k)(dstk3, ones_c, zeros_c)

    xsg_lo, xsg_hi, xsk_lo, xsk_hi = _scale(xw1, degg, degk)
    gs_h = _sc_gs(nbt_g, HH)
    gs_hk = _sc_gs(nbt_k, HH)
    aglo = gs_h(xsg_lo, srcg3, dstg3, zeros_h)
    aghi = gs_h(xsg_hi, srcg3, dstg3, zeros_h)
    aklo = gs_hk(xsk_lo, srck3, dstk3, zeros_h)
    akhi = gs_hk(xsk_hi, srck3, dstk3, zeros_h)

    rw2, xs2g, xs2k = _assemble(aglo, aghi, aklo, akhi, xw1, degg, degk,
                                b1.reshape(1, H), W2)
    acc2g = _sc_gs(nbt_g, C)(xs2g, srcg3, dstg3, zeros_c)
    acc2k = _sc_gs(nbt_k, C)(xs2k, srck3, dstk3, zeros_c)

    return _final(acc2g, acc2k, rw2, degg, degk,
                  b2.reshape(1, C), Wlin, blin.reshape(1, C))


# speedup vs baseline: 1.1611x; 1.1567x over previous
import functools

import jax
import jax.numpy as jnp
from jax import lax
from jax.experimental import pallas as pl
from jax.experimental.pallas import tpu as pltpu
from jax.experimental.pallas import tpu_sc as plsc

N = 10000
F_IN = 128
H = 128
C = 16
KNN = 5
E = 320000

NC = 2
NS = 16
NW = NC * NS
EB = 128
ACC = 10240
TRASH = 10016
BM = 512
KBM = 256
NPADC = 10240


def _cdiv(a, b):
    return (a + b - 1) // b


def _prep_body(x_ref, w1_ref, xn_ref, xw_ref):
    x = x_ref[...]
    nrm = jnp.sqrt(jnp.sum(x * x, axis=1, keepdims=True))
    xn_ref[...] = x / (nrm + 1e-12)
    xw_ref[...] = jnp.dot(x, w1_ref[...], preferred_element_type=jnp.float32)


def _prep(x, W1):
    g = _cdiv(N, BM)
    return pl.pallas_call(
        _prep_body,
        grid=(g,),
        in_specs=[pl.BlockSpec((BM, F_IN), lambda i: (i, 0)),
                  pl.BlockSpec((F_IN, H), lambda i: (0, 0))],
        out_specs=[pl.BlockSpec((BM, F_IN), lambda i: (i, 0)),
                   pl.BlockSpec((BM, H), lambda i: (i, 0))],
        out_shape=[jax.ShapeDtypeStruct((N, F_IN), jnp.float32),
                   jax.ShapeDtypeStruct((N, H), jnp.float32)],
    )(x, W1)


def _knn_body(xn_ref, xnt_ref, out_ref):
    i = pl.program_id(0)
    a = xn_ref[...]
    s = jnp.dot(a, xnt_ref[...], preferred_element_type=jnp.float32)
    rows = i * KBM + lax.broadcasted_iota(jnp.int32, s.shape, 0)
    cols = lax.broadcasted_iota(jnp.int32, s.shape, 1)
    s = jnp.where((cols == rows) | (cols >= N), -1e30, s)
    lane = lax.broadcasted_iota(jnp.int32, (KBM, 128), 1)
    res = jnp.zeros((KBM, 128), jnp.int32)
    for j in range(KNN):
        m = jnp.max(s, axis=1, keepdims=True)
        arg = jnp.min(jnp.where(s == m, cols, jnp.int32(2 ** 30)), axis=1,
                      keepdims=True)
        res = jnp.where(lane == j, arg, res)
        s = jnp.where(cols == arg, -1e30, s)
    out_ref[...] = res


def _knn(xn, xnt_pad):
    g = _cdiv(N, KBM)
    return pl.pallas_call(
        _knn_body,
        grid=(g,),
        in_specs=[pl.BlockSpec((KBM, F_IN), lambda i: (i, 0)),
                  pl.BlockSpec((F_IN, NPADC), lambda i: (0, 0))],
        out_specs=pl.BlockSpec((KBM, 128), lambda i: (i, 0)),
        out_shape=jax.ShapeDtypeStruct((N, 128), jnp.int32),
    )(xn, xnt_pad)


def _dinv(dref):
    d = dref[0, :, 0:1] + dref[1, :, 0:1]
    return jnp.where(d > 0, 1.0 / jnp.sqrt(d), 0.0)


HH = H // 2


def _scale_body(xw_ref, dg_ref, dk_ref, xsg_lo, xsg_hi, xsk_lo, xsk_hi):
    xw = xw_ref[...]
    xsg = xw * _dinv(dg_ref)
    xsk = xw * _dinv(dk_ref)
    xsg_lo[...] = xsg[:, :HH]
    xsg_hi[...] = xsg[:, HH:]
    xsk_lo[...] = xsk[:, :HH]
    xsk_hi[...] = xsk[:, HH:]


def _scale(xw1, degg, degk):
    g = _cdiv(N, BM)
    dspec = pl.BlockSpec((2, BM, C), lambda i: (0, i, 0))
    ospec = pl.BlockSpec((BM, HH), lambda i: (i, 0))
    return pl.pallas_call(
        _scale_body,
        grid=(g,),
        in_specs=[pl.BlockSpec((BM, H), lambda i: (i, 0)), dspec, dspec],
        out_specs=[ospec, ospec, ospec, ospec],
        out_shape=[jax.ShapeDtypeStruct((N, HH), jnp.float32)] * 4,
    )(xw1, degg, degk)


def _assemble_body(aglo_ref, aghi_ref, aklo_ref, akhi_ref, xw_ref, dg_ref,
                   dk_ref, b1_ref, w2_ref, rw2_ref, xs2g_ref, xs2k_ref):
    dig = _dinv(dg_ref)
    dik = _dinv(dk_ref)
    b1 = b1_ref[...]
    ag = jnp.concatenate([aglo_ref[0] + aglo_ref[1],
                          aghi_ref[0] + aghi_ref[1]], axis=1)
    ak = jnp.concatenate([aklo_ref[0] + aklo_ref[1],
                          akhi_ref[0] + akhi_ref[1]], axis=1)
    h1 = jnp.maximum(ag * dig + b1, 0.0)
    h12 = jnp.maximum(ak * dik + b1, 0.0)
    h13 = jnp.maximum(xw_ref[...] + b1, 0.0)
    w2 = w2_ref[...]
    rw2 = (jnp.dot(h1, w2[0:H], preferred_element_type=jnp.float32)
           + jnp.dot(h12, w2[H:2 * H], preferred_element_type=jnp.float32)
           + jnp.dot(h13, w2[2 * H:3 * H], preferred_element_type=jnp.float32))
    rw2_ref[...] = rw2
    xs2g_ref[...] = rw2 * dig
    xs2k_ref[...] = rw2 * dik


def _assemble(aglo, aghi, aklo, akhi, xw1, degg, degk, b1r, W2):
    g = _cdiv(N, BM)
    aspec = pl.BlockSpec((2, BM, HH), lambda i: (0, i, 0))
    dspec = pl.BlockSpec((2, BM, C), lambda i: (0, i, 0))
    ospec = pl.BlockSpec((BM, C), lambda i: (i, 0))
    return pl.pallas_call(
        _assemble_body,
        grid=(g,),
        in_specs=[aspec, aspec, aspec, aspec,
                  pl.BlockSpec((BM, H), lambda i: (i, 0)),
                  dspec, dspec,
                  pl.BlockSpec((1, H), lambda i: (0, 0)),
                  pl.BlockSpec((3 * H, C), lambda i: (0, 0))],
        out_specs=[ospec, ospec, ospec],
        out_shape=[jax.ShapeDtypeStruct((N, C), jnp.float32)] * 3,
    )(aglo, aghi, aklo, akhi, xw1, degg, degk, b1r, W2)


def _final_body(ag_ref, ak_ref, rw2_ref, dg_ref, dk_ref, b2_ref, wl_ref,
                bl_ref, out_ref):
    dig = _dinv(dg_ref)
    dik = _dinv(dk_ref)
    b2 = b2_ref[...]
    h2 = (ag_ref[0] + ag_ref[1]) * dig + b2
    h22 = (ak_ref[0] + ak_ref[1]) * dik + b2
    h23 = rw2_ref[...] + b2
    wl = wl_ref[...]
    f = (jnp.dot(h2, wl[0:C], preferred_element_type=jnp.float32)
         + jnp.dot(h22, wl[C:2 * C], preferred_element_type=jnp.float32)
         + jnp.dot(h23, wl[2 * C:3 * C], preferred_element_type=jnp.float32)
         + bl_ref[...])
    m = jnp.max(f, axis=1, keepdims=True)
    e = jnp.exp(f - m)
    lse = jnp.log(jnp.sum(e, axis=1, keepdims=True))
    out_ref[...] = f - m - lse


def _final(acc2g, acc2k, rw2, degg, degk, b2r, Wlin, blinr):
    g = _cdiv(N, BM)
    aspec = pl.BlockSpec((2, BM, C), lambda i: (0, i, 0))
    return pl.pallas_call(
        _final_body,
        grid=(g,),
        in_specs=[aspec, aspec, pl.BlockSpec((BM, C), lambda i: (i, 0)),
                  aspec, aspec,
                  pl.BlockSpec((1, C), lambda i: (0, 0)),
                  pl.BlockSpec((3 * C, C), lambda i: (0, 0)),
                  pl.BlockSpec((1, C), lambda i: (0, 0))],
        out_specs=pl.BlockSpec((BM, C), lambda i: (i, 0)),
        out_shape=jax.ShapeDtypeStruct((N, C), jnp.float32),
    )(acc2g, acc2k, rw2, degg, degk, b2r, Wlin, blinr)


@functools.lru_cache(maxsize=None)
def _mesh():
    return plsc.VectorSubcoreMesh(core_axis_name="c", subcore_axis_name="s")


_RPT = ACC // NS
_RCH = _RPT // 128


@functools.lru_cache(maxsize=None)
def _sc_hist(nbt):
    np_ = nbt - 1

    @functools.partial(
        pl.kernel,
        mesh=_mesh(),
        compiler_params=pltpu.CompilerParams(use_tc_tiling_on_sc=False),
        out_type=jax.ShapeDtypeStruct((NC, ACC, C), jnp.float32),
        scratch_types=[pltpu.VMEM((nbt, EB), jnp.int32),
                       pltpu.VMEM((128, C), jnp.float32),
                       pltpu.VMEM((128, C), jnp.float32),
                       pltpu.VMEM_SHARED((ACC, C), jnp.float32)],
    )
    def k(dst_hbm, ones_hbm, zeros_hbm, out_hbm, dst_v, ones_v, buf_v, acc):
        c = lax.axis_index("c")
        s = lax.axis_index("s")
        pltpu.sync_copy(dst_hbm.at[c, s], dst_v)
        pltpu.sync_copy(ones_hbm, ones_v)
        pltpu.sync_copy(zeros_hbm, buf_v)
        for t in range(_RCH):
            pltpu.sync_copy(buf_v, acc.at[pl.ds(s * _RPT + t * 128, 128)])
        plsc.subcore_barrier()

        def body(j, carry):
            pltpu.sync_copy(ones_v, acc.at[dst_v.at[j]], add=True)
            return carry

        lax.fori_loop(0, np_, body, 0)
        plsc.subcore_barrier()
        for t in range(_RCH):
            pltpu.sync_copy(acc.at[pl.ds(s * _RPT + t * 128, 128)], buf_v)
            pltpu.sync_copy(buf_v, out_hbm.at[c, pl.ds(s * _RPT + t * 128, 128)])

    return k


@functools.lru_cache(maxsize=None)
def _sc_gs(nbt, d):
    np_ = nbt - 1

    @functools.partial(
        pl.kernel,
        mesh=_mesh(),
        compiler_params=pltpu.CompilerParams(use_tc_tiling_on_sc=False),
        out_type=jax.ShapeDtypeStruct((NC, ACC, d), jnp.float32),
        scratch_types=[pltpu.VMEM((nbt, EB), jnp.int32),
                       pltpu.VMEM((nbt, EB), jnp.int32),
                       pltpu.VMEM((EB, d), jnp.float32),
                       pltpu.VMEM((128, d), jnp.float32),
                       pltpu.VMEM_SHARED((ACC, d), jnp.float32),
                       pltpu.SemaphoreType.DMA],
    )
    def k(tab_hbm, src_hbm, dst_hbm, zeros_hbm, out_hbm,
          src_v, dst_v, rows0, buf_v, acc, sg0):
        c = lax.axis_index("c")
        s = lax.axis_index("s")
        pltpu.sync_copy(src_hbm.at[c, s], src_v)
        pltpu.sync_copy(dst_hbm.at[c, s], dst_v)
        pltpu.sync_copy(zeros_hbm, buf_v)
        for t in range(_RCH):
            pltpu.sync_copy(buf_v, acc.at[pl.ds(s * _RPT + t * 128, 128)])
        plsc.subcore_barrier()

        def body(j, carry):
            pltpu.async_copy(tab_hbm.at[src_v.at[j]], rows0, sg0).wait()
            pltpu.sync_copy(rows0, acc.at[dst_v.at[j]], add=True)
            return carry

        lax.fori_loop(0, np_, body, 0)
        plsc.subcore_barrier()
        for t in range(_RCH):
            pltpu.sync_copy(acc.at[pl.ds(s * _RPT + t * 128, 128)], buf_v)
            pltpu.sync_copy(buf_v, out_hbm.at[c, pl.ds(s * _RPT + t * 128, 128)])

    return k


def _trash(n):
    return N + (jnp.arange(n, dtype=jnp.int32) % (ACC - N))


def _edge_blocks(idx, np_, trash_fill):
    tot = NW * np_ * EB
    pad = tot - idx.shape[0]
    fill = _trash(pad) if trash_fill else jnp.zeros((pad,), jnp.int32)
    extra = (_trash(NW * EB) if trash_fill
             else jnp.zeros((NW * EB,), jnp.int32)).reshape(NW, EB)
    arr = jnp.concatenate([idx, fill]).reshape(NW, np_ * EB)
    arr = jnp.concatenate([arr, extra], axis=1)
    return arr.reshape(NC, NS, np_ + 1, EB)


def kernel(x, edge_index, W1, b1, W2, b2, Wlin, blin):
    f32 = jnp.float32
    np_g = 80
    np_k = 14
    nbt_g = np_g + 1
    nbt_k = np_k + 1

    xn, xw1 = _prep(x, W1)
    xnt_pad = jnp.concatenate(
        [xn.T, jnp.zeros((F_IN, NPADC - N), f32)], axis=1)
    nbr = _knn(xn, xnt_pad)[:, :KNN]

    src_g = edge_index[0]
    dst_g = jnp.where(edge_index[0] == edge_index[1],
                      N + (edge_index[0] % (ACC - N)), edge_index[1])
    srcg3 = _edge_blocks(src_g, np_g, False)
    dstg3 = _edge_blocks(dst_g, np_g, True)
    src_k = jnp.broadcast_to(jnp.arange(N, dtype=jnp.int32)[:, None],
                             (N, KNN)).reshape(-1)
    dst_k = nbr.reshape(-1)
    srck3 = _edge_blocks(src_k, np_k, False)
    dstk3 = _edge_blocks(dst_k, np_k, True)

    ones_c = jnp.ones((128, C), f32)
    zeros_c = jnp.zeros((128, C), f32)
    zeros_h = jnp.zeros((128, HH), f32)

    degg = _sc_hist(nbt_g)(dstg3, ones_c, zeros_c)
    degk = _sc_hist(nbt_k)(dstk3, ones_c, zeros_c)

    xsg_lo, xsg_hi, xsk_lo, xsk_hi = _scale(xw1, degg, degk)
    gs_h = _sc_gs(nbt_g, HH)
    gs_hk = _sc_gs(nbt_k, HH)
    aglo = gs_h(xsg_lo, srcg3, dstg3, zeros_h)
    aghi = gs_h(xsg_hi, srcg3, dstg3, zeros_h)
    aklo = gs_hk(xsk_lo, srck3, dstk3, zeros_h)
    akhi = gs_hk(xsk_hi, srck3, dstk3, zeros_h)

    rw2, xs2g, xs2k = _assemble(aglo, aghi, aklo, akhi, xw1, degg, degk,
                                b1.reshape(1, H), W2)
    acc2g = _sc_gs(nbt_g, C)(xs2g, srcg3, dstg3, zeros_c)
    acc2k = _sc_gs(nbt_k, C)(xs2k, srck3, dstk3, zeros_c)

    return _final(acc2g, acc2k, rw2, degg, degk,
                  b2.reshape(1, C), Wlin, blin.reshape(1, C))


# speedup vs baseline: 1.3729x; 1.1823x over previous
import functools

import jax
import jax.numpy as jnp
from jax import lax
from jax.experimental import pallas as pl
from jax.experimental.pallas import tpu as pltpu
from jax.experimental.pallas import tpu_sc as plsc

N = 10000
F_IN = 128
H = 128
C = 16
KNN = 5
E = 320000

NC = 2
NS = 16
NW = NC * NS
EB = 128
ACC = 10240
TRASH = 10016
BM = 512
KBM = 256
NPADC = 10240


def _cdiv(a, b):
    return (a + b - 1) // b


def _prep_body(x_ref, w1_ref, xn_ref, xw_ref):
    x = x_ref[...]
    nrm = jnp.sqrt(jnp.sum(x * x, axis=1, keepdims=True))
    xn_ref[...] = x / (nrm + 1e-12)
    xw_ref[...] = jnp.dot(x, w1_ref[...], preferred_element_type=jnp.float32)


def _prep(x, W1):
    g = _cdiv(N, BM)
    return pl.pallas_call(
        _prep_body,
        grid=(g,),
        in_specs=[pl.BlockSpec((BM, F_IN), lambda i: (i, 0)),
                  pl.BlockSpec((F_IN, H), lambda i: (0, 0))],
        out_specs=[pl.BlockSpec((BM, F_IN), lambda i: (i, 0)),
                   pl.BlockSpec((BM, H), lambda i: (i, 0))],
        out_shape=[jax.ShapeDtypeStruct((N, F_IN), jnp.float32),
                   jax.ShapeDtypeStruct((N, H), jnp.float32)],
    )(x, W1)


def _knn_body(xn_ref, xnt_ref, out_ref):
    i = pl.program_id(0)
    a = xn_ref[...]
    s = jnp.dot(a, xnt_ref[...], preferred_element_type=jnp.float32)
    rows = i * KBM + lax.broadcasted_iota(jnp.int32, s.shape, 0)
    cols = lax.broadcasted_iota(jnp.int32, s.shape, 1)
    s = jnp.where((cols == rows) | (cols >= N), -1e30, s)
    lane = lax.broadcasted_iota(jnp.int32, (KBM, 128), 1)
    res = jnp.zeros((KBM, 128), jnp.int32)
    for j in range(KNN):
        m = jnp.max(s, axis=1, keepdims=True)
        arg = jnp.min(jnp.where(s == m, cols, jnp.int32(2 ** 30)), axis=1,
                      keepdims=True)
        res = jnp.where(lane == j, arg, res)
        s = jnp.where(cols == arg, -1e30, s)
    out_ref[...] = res


def _knn(xn, xnt_pad):
    g = _cdiv(N, KBM)
    return pl.pallas_call(
        _knn_body,
        grid=(g,),
        in_specs=[pl.BlockSpec((KBM, F_IN), lambda i: (i, 0)),
                  pl.BlockSpec((F_IN, NPADC), lambda i: (0, 0))],
        out_specs=pl.BlockSpec((KBM, 128), lambda i: (i, 0)),
        out_shape=jax.ShapeDtypeStruct((N, 128), jnp.int32),
    )(xn, xnt_pad)


def _dinv(dref):
    d = dref[0, :, 0:1] + dref[1, :, 0:1]
    return jnp.where(d > 0, 1.0 / jnp.sqrt(d), 0.0)


HH = H // 2


def _scale_body(xw_ref, dg_ref, dk_ref, xsg_lo, xsg_hi, xsk_lo, xsk_hi):
    xw = xw_ref[...]
    xsg = xw * _dinv(dg_ref)
    xsk = xw * _dinv(dk_ref)
    xsg_lo[...] = xsg[:, :HH]
    xsg_hi[...] = xsg[:, HH:]
    xsk_lo[...] = xsk[:, :HH]
    xsk_hi[...] = xsk[:, HH:]


def _scale(xw1, degg, degk):
    g = _cdiv(N, BM)
    dspec = pl.BlockSpec((2, BM, C), lambda i: (0, i, 0))
    ospec = pl.BlockSpec((BM, HH), lambda i: (i, 0))
    return pl.pallas_call(
        _scale_body,
        grid=(g,),
        in_specs=[pl.BlockSpec((BM, H), lambda i: (i, 0)), dspec, dspec],
        out_specs=[ospec, ospec, ospec, ospec],
        out_shape=[jax.ShapeDtypeStruct((N, HH), jnp.float32)] * 4,
    )(xw1, degg, degk)


def _assemble_body(aglo_ref, aghi_ref, aklo_ref, akhi_ref, xw_ref, dg_ref,
                   dk_ref, b1_ref, w2_ref, rw2_ref, xs2g_ref, xs2k_ref):
    dig = _dinv(dg_ref)
    dik = _dinv(dk_ref)
    b1 = b1_ref[...]
    ag = jnp.concatenate([aglo_ref[0] + aglo_ref[1],
                          aghi_ref[0] + aghi_ref[1]], axis=1)
    ak = jnp.concatenate([aklo_ref[0] + aklo_ref[1],
                          akhi_ref[0] + akhi_ref[1]], axis=1)
    h1 = jnp.maximum(ag * dig + b1, 0.0)
    h12 = jnp.maximum(ak * dik + b1, 0.0)
    h13 = jnp.maximum(xw_ref[...] + b1, 0.0)
    w2 = w2_ref[...]
    rw2 = (jnp.dot(h1, w2[0:H], preferred_element_type=jnp.float32)
           + jnp.dot(h12, w2[H:2 * H], preferred_element_type=jnp.float32)
           + jnp.dot(h13, w2[2 * H:3 * H], preferred_element_type=jnp.float32))
    rw2_ref[...] = rw2
    xs2g_ref[...] = rw2 * dig
    xs2k_ref[...] = rw2 * dik


def _assemble(aglo, aghi, aklo, akhi, xw1, degg, degk, b1r, W2):
    g = _cdiv(N, BM)
    aspec = pl.BlockSpec((2, BM, HH), lambda i: (0, i, 0))
    dspec = pl.BlockSpec((2, BM, C), lambda i: (0, i, 0))
    ospec = pl.BlockSpec((BM, C), lambda i: (i, 0))
    return pl.pallas_call(
        _assemble_body,
        grid=(g,),
        in_specs=[aspec, aspec, aspec, aspec,
                  pl.BlockSpec((BM, H), lambda i: (i, 0)),
                  dspec, dspec,
                  pl.BlockSpec((1, H), lambda i: (0, 0)),
                  pl.BlockSpec((3 * H, C), lambda i: (0, 0))],
        out_specs=[ospec, ospec, ospec],
        out_shape=[jax.ShapeDtypeStruct((N, C), jnp.float32)] * 3,
    )(aglo, aghi, aklo, akhi, xw1, degg, degk, b1r, W2)


def _final_body(ag_ref, ak_ref, rw2_ref, dg_ref, dk_ref, b2_ref, wl_ref,
                bl_ref, out_ref):
    dig = _dinv(dg_ref)
    dik = _dinv(dk_ref)
    b2 = b2_ref[...]
    h2 = (ag_ref[0] + ag_ref[1]) * dig + b2
    h22 = (ak_ref[0] + ak_ref[1]) * dik + b2
    h23 = rw2_ref[...] + b2
    wl = wl_ref[...]
    f = (jnp.dot(h2, wl[0:C], preferred_element_type=jnp.float32)
         + jnp.dot(h22, wl[C:2 * C], preferred_element_type=jnp.float32)
         + jnp.dot(h23, wl[2 * C:3 * C], preferred_element_type=jnp.float32)
         + bl_ref[...])
    m = jnp.max(f, axis=1, keepdims=True)
    e = jnp.exp(f - m)
    lse = jnp.log(jnp.sum(e, axis=1, keepdims=True))
    out_ref[...] = f - m - lse


def _final(acc2g, acc2k, rw2, degg, degk, b2r, Wlin, blinr):
    g = _cdiv(N, BM)
    aspec = pl.BlockSpec((2, BM, C), lambda i: (0, i, 0))
    return pl.pallas_call(
        _final_body,
        grid=(g,),
        in_specs=[aspec, aspec, pl.BlockSpec((BM, C), lambda i: (i, 0)),
                  aspec, aspec,
                  pl.BlockSpec((1, C), lambda i: (0, 0)),
                  pl.BlockSpec((3 * C, C), lambda i: (0, 0)),
                  pl.BlockSpec((1, C), lambda i: (0, 0))],
        out_specs=pl.BlockSpec((BM, C), lambda i: (i, 0)),
        out_shape=jax.ShapeDtypeStruct((N, C), jnp.float32),
    )(acc2g, acc2k, rw2, degg, degk, b2r, Wlin, blinr)


@functools.lru_cache(maxsize=None)
def _mesh():
    return plsc.VectorSubcoreMesh(core_axis_name="c", subcore_axis_name="s")


_RPT = ACC // NS
_RCH = _RPT // 128


@functools.lru_cache(maxsize=None)
def _sc_hist(nb):
    np_ = nb

    @functools.partial(
        pl.kernel,
        mesh=_mesh(),
        compiler_params=pltpu.CompilerParams(use_tc_tiling_on_sc=False),
        out_type=jax.ShapeDtypeStruct((NC, ACC, C), jnp.float32),
        scratch_types=[pltpu.VMEM((nb, EB), jnp.int32),
                       pltpu.VMEM((128, C), jnp.float32),
                       pltpu.VMEM((128, C), jnp.float32),
                       pltpu.VMEM_SHARED((ACC, C), jnp.float32)],
    )
    def k(dst_hbm, ones_hbm, zeros_hbm, out_hbm, dst_v, ones_v, buf_v, acc):
        c = lax.axis_index("c")
        s = lax.axis_index("s")
        pltpu.sync_copy(dst_hbm.at[c, s], dst_v)
        pltpu.sync_copy(ones_hbm, ones_v)
        pltpu.sync_copy(zeros_hbm, buf_v)
        for t in range(_RCH):
            pltpu.sync_copy(buf_v, acc.at[pl.ds(s * _RPT + t * 128, 128)])
        plsc.subcore_barrier()

        def body(j, carry):
            pltpu.sync_copy(ones_v, acc.at[dst_v.at[j]], add=True)
            return carry

        lax.fori_loop(0, np_, body, 0)
        plsc.subcore_barrier()
        for t in range(_RCH):
            pltpu.sync_copy(acc.at[pl.ds(s * _RPT + t * 128, 128)], buf_v)
            pltpu.sync_copy(buf_v, out_hbm.at[c, pl.ds(s * _RPT + t * 128, 128)])

    return k


@functools.lru_cache(maxsize=None)
def _sc_gs(nb, d):
    np_ = nb

    @functools.partial(
        pl.kernel,
        mesh=_mesh(),
        compiler_params=pltpu.CompilerParams(use_tc_tiling_on_sc=False),
        out_type=jax.ShapeDtypeStruct((NC, ACC, d), jnp.float32),
        scratch_types=[pltpu.VMEM((nb, EB), jnp.int32),
                       pltpu.VMEM((nb, EB), jnp.int32),
                       pltpu.VMEM((EB, d), jnp.float32),
                       pltpu.VMEM((128, d), jnp.float32),
                       pltpu.VMEM_SHARED((ACC, d), jnp.float32),
                       pltpu.SemaphoreType.DMA],
    )
    def k(tab_hbm, src_hbm, dst_hbm, zeros_hbm, out_hbm,
          src_v, dst_v, rows0, buf_v, acc, sg0):
        c = lax.axis_index("c")
        s = lax.axis_index("s")
        pltpu.sync_copy(src_hbm.at[c, s], src_v)
        pltpu.sync_copy(dst_hbm.at[c, s], dst_v)
        pltpu.sync_copy(zeros_hbm, buf_v)
        for t in range(_RCH):
            pltpu.sync_copy(buf_v, acc.at[pl.ds(s * _RPT + t * 128, 128)])
        plsc.subcore_barrier()

        def body(j, carry):
            pltpu.async_copy(tab_hbm.at[src_v.at[j]], rows0, sg0).wait()
            pltpu.sync_copy(rows0, acc.at[dst_v.at[j]], add=True)
            return carry

        lax.fori_loop(0, np_, body, 0)
        plsc.subcore_barrier()
        for t in range(_RCH):
            pltpu.sync_copy(acc.at[pl.ds(s * _RPT + t * 128, 128)], buf_v)
            pltpu.sync_copy(buf_v, out_hbm.at[c, pl.ds(s * _RPT + t * 128, 128)])

    return k


def _edge_blocks(idx, fill, nb):
    tot = NW * nb * EB
    pad = tot - idx.shape[0]
    idx = jnp.concatenate(
        [idx, jnp.full((pad,), fill, jnp.int32)]) if pad else idx
    return idx.reshape(NC, NS, nb, EB)


def kernel(x, edge_index, W1, b1, W2, b2, Wlin, blin):
    f32 = jnp.float32
    nb_g = _cdiv(_cdiv(E, NW), EB)
    nb_k = _cdiv(_cdiv(N * KNN, NW), EB)

    xn, xw1 = _prep(x, W1)
    xnt_pad = jnp.concatenate(
        [xn.T, jnp.zeros((F_IN, NPADC - N), f32)], axis=1)
    nbr = _knn(xn, xnt_pad)[:, :KNN]

    src_g = edge_index[0]
    dst_g = jnp.where(edge_index[0] == edge_index[1], TRASH, edge_index[1])
    srcg3 = _edge_blocks(src_g, 0, nb_g)
    dstg3 = _edge_blocks(dst_g, TRASH, nb_g)
    src_k = jnp.broadcast_to(jnp.arange(N, dtype=jnp.int32)[:, None],
                             (N, KNN)).reshape(-1)
    dst_k = nbr.reshape(-1)
    srck3 = _edge_blocks(src_k, 0, nb_k)
    dstk3 = _edge_blocks(dst_k, TRASH, nb_k)

    ones_c = jnp.ones((128, C), f32)
    zeros_c = jnp.zeros((128, C), f32)
    zeros_h = jnp.zeros((128, HH), f32)

    degg = _sc_hist(nb_g)(dstg3, ones_c, zeros_c)
    degk = _sc_hist(nb_k)(dstk3, ones_c, zeros_c)

    xsg_lo, xsg_hi, xsk_lo, xsk_hi = _scale(xw1, degg, degk)
    gs_h = _sc_gs(nb_g, HH)
    gs_hk = _sc_gs(nb_k, HH)
    aglo = gs_h(xsg_lo, srcg3, dstg3, zeros_h)
    aghi = gs_h(xsg_hi, srcg3, dstg3, zeros_h)
    aklo = gs_hk(xsk_lo, srck3, dstk3, zeros_h)
    akhi = gs_hk(xsk_hi, srck3, dstk3, zeros_h)

    rw2, xs2g, xs2k = _assemble(aglo, aghi, aklo, akhi, xw1, degg, degk,
                                b1.reshape(1, H), W2)
    acc2g = _sc_gs(nb_g, C)(xs2g, srcg3, dstg3, zeros_c)
    acc2k = _sc_gs(nb_k, C)(xs2k, srck3, dstk3, zeros_c)

    return _final(acc2g, acc2k, rw2, degg, degk,
                  b2.reshape(1, C), Wlin, blin.reshape(1, C))
